# baseline (speedup 1.0000x reference)
import jax
import jax.numpy as jnp
from jax.experimental import pallas as pl
from jax.experimental.pallas import tpu as pltpu

_FREQ_MASK_PARAM = 27
_TIME_MASK_PARAM = 100
_N_FREQ_MASKS = 2
_N_TIME_MASKS = 2
_TIME_MASK_RATIO = 0.05

_LANES = 640
_ROWS = 375


def _mask_body(s_ref, x_ref, o_ref):
    b = pl.program_id(0)
    _, rows, lanes = x_ref.shape
    lane = jax.lax.broadcasted_iota(jnp.int32, (1, 1, lanes), 2)
    rowbase = jax.lax.broadcasted_iota(jnp.int32, (1, rows, 1), 1) * lanes
    pos = rowbase + lane
    tmask = ((pos >= s_ref[4, b]) & (pos < s_ref[5, b])) | (
        (pos >= s_ref[6, b]) & (pos < s_ref[7, b]))
    f = lane % 80
    fmask = ((f >= s_ref[0, b]) & (f < s_ref[1, b])) | (
        (f >= s_ref[2, b]) & (f < s_ref[3, b]))
    o_ref[...] = jnp.where(tmask | fmask, jnp.float32(0.0), x_ref[...])


def _mask_bounds(B, T, F):
    key = jax.random.key(42)
    rows = []
    for _ in range(_N_FREQ_MASKS):
        key, k1, k2 = jax.random.split(key, 3)
        f = jax.random.randint(k1, (B,), 0, _FREQ_MASK_PARAM + 1)
        f0 = jax.random.randint(k2, (B,), 0, max(1, F - _FREQ_MASK_PARAM))
        rows += [f0, f0 + f]
    t_max = _TIME_MASK_PARAM
    if _TIME_MASK_RATIO is not None:
        t_max = min(t_max, int(_TIME_MASK_RATIO * T))
    for _ in range(_N_TIME_MASKS):
        key, k1, k2 = jax.random.split(key, 3)
        t = jax.random.randint(k1, (B,), 0, max(1, t_max + 1))
        t0 = jax.random.randint(k2, (B,), 0, max(1, T - t_max))
        rows += [t0 * F, (t0 + t) * F]
    return jnp.stack(rows).astype(jnp.int32)


def kernel(x):
    B, T, F = x.shape
    bounds = _mask_bounds(B, T, F)
    xf = x.reshape(B, _ROWS, _LANES)
    out = pl.pallas_call(
        _mask_body,
        grid=(B,),
        in_specs=[
            pl.BlockSpec(memory_space=pltpu.SMEM),
            pl.BlockSpec((1, _ROWS, _LANES), lambda b: (b, 0, 0)),
        ],
        out_specs=pl.BlockSpec((1, _ROWS, _LANES), lambda b: (b, 0, 0)),
        out_shape=jax.ShapeDtypeStruct((B, _ROWS, _LANES), x.dtype),
    )(bounds, xf)
    return out.reshape(B, T, F)


# speedup vs baseline: 1.7640x; 1.7640x over previous
import jax
import jax.numpy as jnp
from jax.experimental import pallas as pl
from jax.experimental.pallas import tpu as pltpu

_FREQ_MASK_PARAM = 27
_TIME_MASK_PARAM = 100
_N_FREQ_MASKS = 2
_N_TIME_MASKS = 2
_TIME_MASK_RATIO = 0.05

_K = 8


def _mask_body(s_ref, x_hbm, o_hbm, in_buf, out_buf, in_sem, out_sem):
    B, T, F = x_hbm.shape

    def in_copy(b, k):
        return pltpu.make_async_copy(x_hbm.at[b], in_buf.at[k], in_sem.at[k])

    def out_copy(b, k):
        return pltpu.make_async_copy(out_buf.at[k], o_hbm.at[b], out_sem.at[k])

    for k in range(_K):
        in_copy(k, k).start()

    rows = jax.lax.broadcasted_iota(jnp.int32, (1, T, 1), 1)
    cols = jax.lax.broadcasted_iota(jnp.int32, (1, 1, F), 2)

    def step(b, carry):
        k = jax.lax.rem(b, _K)
        in_copy(b, k).wait()

        @pl.when(b >= _K)
        def _():
            out_copy(b - _K, k).wait()

        tmask = ((rows >= s_ref[4, b]) & (rows < s_ref[5, b])) | (
            (rows >= s_ref[6, b]) & (rows < s_ref[7, b]))
        fmask = ((cols >= s_ref[0, b]) & (cols < s_ref[1, b])) | (
            (cols >= s_ref[2, b]) & (cols < s_ref[3, b]))
        out_buf[pl.ds(k, 1)] = jnp.where(
            tmask | fmask, jnp.float32(0.0), in_buf[pl.ds(k, 1)])
        out_copy(b, k).start()

        @pl.when(b + _K < B)
        def _():
            in_copy(b + _K, k).start()

        return carry

    jax.lax.fori_loop(0, B, step, 0)
    for k in range(_K):
        out_copy(B - _K + k, (B - _K + k) % _K).wait()


def _mask_bounds(B, T, F):
    key = jax.random.key(42)
    rows = []
    for _ in range(_N_FREQ_MASKS):
        key, k1, k2 = jax.random.split(key, 3)
        f = jax.random.randint(k1, (B,), 0, _FREQ_MASK_PARAM + 1)
        f0 = jax.random.randint(k2, (B,), 0, max(1, F - _FREQ_MASK_PARAM))
        rows += [f0, f0 + f]
    t_max = _TIME_MASK_PARAM
    if _TIME_MASK_RATIO is not None:
        t_max = min(t_max, int(_TIME_MASK_RATIO * T))
    for _ in range(_N_TIME_MASKS):
        key, k1, k2 = jax.random.split(key, 3)
        t = jax.random.randint(k1, (B,), 0, max(1, t_max + 1))
        t0 = jax.random.randint(k2, (B,), 0, max(1, T - t_max))
        rows += [t0, t0 + t]
    return jnp.stack(rows).astype(jnp.int32)


def kernel(x):
    B, T, F = x.shape
    bounds = _mask_bounds(B, T, F)
    return pl.pallas_call(
        _mask_body,
        in_specs=[
            pl.BlockSpec(memory_space=pltpu.SMEM),
            pl.BlockSpec(memory_space=pltpu.HBM),
        ],
        out_specs=pl.BlockSpec(memory_space=pltpu.HBM),
        out_shape=jax.ShapeDtypeStruct((B, T, F), x.dtype),
        scratch_shapes=[
            pltpu.VMEM((_K, T, F), jnp.float32),
            pltpu.VMEM((_K, T, F), jnp.float32),
            pltpu.SemaphoreType.DMA((_K,)),
            pltpu.SemaphoreType.DMA((_K,)),
        ],
    )(bounds, x)


# speedup vs baseline: 1.7698x; 1.0033x over previous
import jax
import jax.numpy as jnp
from jax.experimental import pallas as pl
from jax.experimental.pallas import tpu as pltpu

_FREQ_MASK_PARAM = 27
_TIME_MASK_PARAM = 100
_N_FREQ_MASKS = 2
_N_TIME_MASKS = 2
_TIME_MASK_RATIO = 0.05

_H = 2
_K = 16


def _mask_body(s_ref, x_hbm, o_hbm, in_buf, out_buf, in_sem, out_sem):
    B, T, F = x_hbm.shape
    R = T // _H
    C = B * _H

    def in_copy(c, k):
        b, h = jax.lax.div(c, _H), jax.lax.rem(c, _H)
        return pltpu.make_async_copy(
            x_hbm.at[b, pl.ds(h * R, R)], in_buf.at[k], in_sem.at[k])

    def out_copy(c, k):
        b, h = jax.lax.div(c, _H), jax.lax.rem(c, _H)
        return pltpu.make_async_copy(
            out_buf.at[k], o_hbm.at[b, pl.ds(h * R, R)], out_sem.at[k])

    for k in range(_K):
        in_copy(jnp.int32(k), k).start()

    rows = jax.lax.broadcasted_iota(jnp.int32, (1, R, 1), 1)
    cols = jax.lax.broadcasted_iota(jnp.int32, (1, 1, F), 2)

    def step(c, carry):
        k = jax.lax.rem(c, _K)
        b, h = jax.lax.div(c, _H), jax.lax.rem(c, _H)
        in_copy(c, k).wait()

        @pl.when(c >= _K)
        def _():
            out_copy(c - _K, k).wait()

        row = rows + h * R
        tmask = ((row >= s_ref[4, b]) & (row < s_ref[5, b])) | (
            (row >= s_ref[6, b]) & (row < s_ref[7, b]))
        fmask = ((cols >= s_ref[0, b]) & (cols < s_ref[1, b])) | (
            (cols >= s_ref[2, b]) & (cols < s_ref[3, b]))
        out_buf[pl.ds(k, 1)] = jnp.where(
            tmask | fmask, jnp.float32(0.0), in_buf[pl.ds(k, 1)])
        out_copy(c, k).start()

        @pl.when(c + _K < C)
        def _():
            in_copy(c + _K, k).start()

        return carry

    jax.lax.fori_loop(0, C, step, 0)
    for k in range(_K):
        out_copy(jnp.int32(C - _K + k), (C - _K + k) % _K).wait()


def _mask_bounds(B, T, F):
    key = jax.random.key(42)
    rows = []
    for _ in range(_N_FREQ_MASKS):
        key, k1, k2 = jax.random.split(key, 3)
        f = jax.random.randint(k1, (B,), 0, _FREQ_MASK_PARAM + 1)
        f0 = jax.random.randint(k2, (B,), 0, max(1, F - _FREQ_MASK_PARAM))
        rows += [f0, f0 + f]
    t_max = _TIME_MASK_PARAM
    if _TIME_MASK_RATIO is not None:
        t_max = min(t_max, int(_TIME_MASK_RATIO * T))
    for _ in range(_N_TIME_MASKS):
        key, k1, k2 = jax.random.split(key, 3)
        t = jax.random.randint(k1, (B,), 0, max(1, t_max + 1))
        t0 = jax.random.randint(k2, (B,), 0, max(1, T - t_max))
        rows += [t0, t0 + t]
    return jnp.stack(rows).astype(jnp.int32)


def kernel(x):
    B, T, F = x.shape
    bounds = _mask_bounds(B, T, F)
    return pl.pallas_call(
        _mask_body,
        in_specs=[
            pl.BlockSpec(memory_space=pltpu.SMEM),
            pl.BlockSpec(memory_space=pltpu.HBM),
        ],
        out_specs=pl.BlockSpec(memory_space=pltpu.HBM),
        out_shape=jax.ShapeDtypeStruct((B, T, F), x.dtype),
        scratch_shapes=[
            pltpu.VMEM((_K, T // _H, F), jnp.float32),
            pltpu.VMEM((_K, T // _H, F), jnp.float32),
            pltpu.SemaphoreType.DMA((_K,)),
            pltpu.SemaphoreType.DMA((_K,)),
        ],
    )(bounds, x)
